# Initial kernel scaffold; baseline (speedup 1.0000x reference)
import jax
import jax.numpy as jnp
from jax.experimental import pallas as pl


def kernel(encoder_out, align_phone, text_phone, pitch, beats, W_pitch, b_pitch, W_pos, b_pos, emb_beats):
    raise NotImplementedError("write your pallas kernel here")


# speedup vs baseline: 10.7977x; 10.7977x over previous
import numpy as np
import jax
import jax.numpy as jnp
from jax.experimental import pallas as pl
from jax.experimental.pallas import tpu as pltpu

B, FRAME, TTEXT, D = 16, 2048, 512, 1024


def _pe_const(length, d):
    pos = np.arange(length, dtype=np.float32)[:, None]
    div = np.exp(np.arange(0, d, 2, dtype=np.float32) * (-np.log(10000.0) / d))
    pe = np.zeros((length, d), dtype=np.float32)
    pe[:, 0::2] = np.sin(pos * div)
    pe[:, 1::2] = np.cos(pos * div)
    return pe


_PE = _pe_const(FRAME, D)


def _scan_body(align_ref, text_ref, idx_ref):
    idx_ref[0, 0, 0] = jnp.int32(0)

    def step(j, carry):
        ind, bt = carry
        aj = align_ref[0, 0, j]
        same = aj == bt
        nind = jnp.where(same, ind, jnp.minimum(ind + 1, TTEXT - 1))
        nbt = jnp.where(same, bt, text_ref[0, 0, nind])
        idx_ref[0, 0, j] = nind
        return nind, nbt

    jax.lax.fori_loop(1, FRAME, step, (jnp.int32(0), text_ref[0, 0, 0]))


def _compute_idx(align3, text3):
    return pl.pallas_call(
        _scan_body,
        grid=(B,),
        in_specs=[
            pl.BlockSpec((1, 1, FRAME), lambda b: (b, 0, 0), memory_space=pltpu.SMEM),
            pl.BlockSpec((1, 1, TTEXT), lambda b: (b, 0, 0), memory_space=pltpu.SMEM),
        ],
        out_specs=pl.BlockSpec((1, 1, FRAME), lambda b: (b, 0, 0), memory_space=pltpu.SMEM),
        out_shape=jax.ShapeDtypeStruct((B, 1, FRAME), jnp.int32),
    )(align3, text3)


def _pep_body(pe_ref, w_ref, bias_ref, out_ref):
    out_ref[...] = (
        jnp.dot(pe_ref[...], w_ref[...], preferred_element_type=jnp.float32)
        + bias_ref[...]
    )


def _compute_pep(pe, w_pos, bias_row):
    blk = 512
    return pl.pallas_call(
        _pep_body,
        grid=(FRAME // blk,),
        in_specs=[
            pl.BlockSpec((blk, D), lambda i: (i, 0)),
            pl.BlockSpec((D, D), lambda i: (0, 0)),
            pl.BlockSpec((1, D), lambda i: (0, 0)),
        ],
        out_specs=pl.BlockSpec((blk, D), lambda i: (i, 0)),
        out_shape=jax.ShapeDtypeStruct((FRAME, D), jnp.float32),
    )(pe, w_pos, bias_row)


def _main_body(enc_ref, w_ref, pep_ref, idx_ref, pitch_ref, beats_ref,
               wp_ref, de_ref, out_ref, s_ref):
    enc = enc_ref[0]
    s_ref[...] = enc + jnp.dot(enc, w_ref[...], preferred_element_type=jnp.float32)

    def row(i, _):
        out_ref[0, i, :] = s_ref[idx_ref[0, 0, i], :]
        return 0

    jax.lax.fori_loop(0, FRAME, row, 0)

    out_ref[0] = (
        out_ref[0]
        + pep_ref[...]
        + pitch_ref[0] * wp_ref[...]
        + beats_ref[0].astype(jnp.float32) * de_ref[...]
    )


def _main(enc, w_pos, pep, idx3, pitch, beats, w_pitch, de_row):
    return pl.pallas_call(
        _main_body,
        grid=(B,),
        in_specs=[
            pl.BlockSpec((1, TTEXT, D), lambda b: (b, 0, 0)),
            pl.BlockSpec((D, D), lambda b: (0, 0)),
            pl.BlockSpec((FRAME, D), lambda b: (0, 0)),
            pl.BlockSpec((1, 1, FRAME), lambda b: (b, 0, 0), memory_space=pltpu.SMEM),
            pl.BlockSpec((1, FRAME, 1), lambda b: (b, 0, 0)),
            pl.BlockSpec((1, FRAME, 1), lambda b: (b, 0, 0)),
            pl.BlockSpec((1, D), lambda b: (0, 0)),
            pl.BlockSpec((1, D), lambda b: (0, 0)),
        ],
        out_specs=pl.BlockSpec((1, FRAME, D), lambda b: (b, 0, 0)),
        out_shape=jax.ShapeDtypeStruct((B, FRAME, D), jnp.float32),
        scratch_shapes=[pltpu.VMEM((TTEXT, D), jnp.float32)],
    )(enc, w_pos, pep, idx3, pitch, beats, w_pitch, de_row)


def kernel(encoder_out, align_phone, text_phone, pitch, beats,
           W_pitch, b_pitch, W_pos, b_pos, emb_beats):
    align3 = align_phone.astype(jnp.int32).reshape(B, 1, FRAME)
    text3 = text_phone.astype(jnp.int32).reshape(B, 1, TTEXT)
    idx3 = _compute_idx(align3, text3)

    pe = jnp.asarray(_PE)
    bias_row = (b_pos + b_pitch + emb_beats[0]).reshape(1, D)
    de_row = (emb_beats[1] - emb_beats[0]).reshape(1, D)
    pep = _compute_pep(pe, W_pos, bias_row)

    return _main(encoder_out, W_pos, pep, idx3, pitch,
                 beats.astype(jnp.int32), W_pitch.reshape(1, D), de_row)


# speedup vs baseline: 19.0886x; 1.7678x over previous
import numpy as np
import jax
import jax.numpy as jnp
from jax.experimental import pallas as pl
from jax.experimental.pallas import tpu as pltpu

B, FRAME, TTEXT, D = 16, 2048, 512, 1024
SBLK = 64
GRP = 8
MCH = 64


def _pe_const(length, d):
    pos = np.arange(length, dtype=np.float32)[:, None]
    div = np.exp(np.arange(0, d, 2, dtype=np.float32) * (-np.log(10000.0) / d))
    pe = np.zeros((length, d), dtype=np.float32)
    pe[:, 0::2] = np.sin(pos * div)
    pe[:, 1::2] = np.cos(pos * div)
    return pe


_PE = _pe_const(FRAME, D)


def _scan_body(alignT_ref, textT_ref, idxT_ref):
    tmax = jnp.int32(TTEXT - 1)

    def gather_text(nind):
        def acc_chunk(m, acc):
            tv = textT_ref[pl.ds(m * 8, 8), :]
            rows = m * 8 + jax.lax.broadcasted_iota(jnp.int32, (8, B), 0)
            hit = rows == nind
            return acc + jnp.sum(jnp.where(hit, tv, 0), axis=0, keepdims=True)
        return jax.lax.fori_loop(0, TTEXT // 8, acc_chunk,
                                 jnp.zeros((1, B), jnp.int32))

    def slow_block(k, ind, bt):
        def step(j, carry):
            ind, bt = carry
            a = alignT_ref[pl.ds(k * SBLK + j, 1), :]
            adv = a != bt
            nind = jnp.where(adv, jnp.minimum(ind + 1, tmax), ind)
            nbt = jnp.where(adv, gather_text(nind), bt)
            idxT_ref[pl.ds(k * SBLK + j, 1), :] = nind
            return nind, nbt

        return jax.lax.fori_loop(0, SBLK, step, (ind, bt))

    def block(k, carry):
        ind, bt = carry
        blk = alignT_ref[pl.ds(k * SBLK, SBLK), :]
        uniform = jnp.all(blk == bt)

        def fast(args):
            ind, bt = args
            idxT_ref[pl.ds(k * SBLK, SBLK), :] = jnp.broadcast_to(ind, (SBLK, B))
            return ind, bt

        return jax.lax.cond(uniform, fast, lambda args: slow_block(k, *args),
                            (ind, bt))

    init = (jnp.zeros((1, B), jnp.int32), textT_ref[0:1, :])
    jax.lax.fori_loop(0, FRAME // SBLK, block, init)


def _compute_idx(alignT, textT):
    return pl.pallas_call(
        _scan_body,
        in_specs=[
            pl.BlockSpec((FRAME, B), lambda: (0, 0)),
            pl.BlockSpec((TTEXT, B), lambda: (0, 0)),
        ],
        out_specs=pl.BlockSpec((FRAME, B), lambda: (0, 0)),
        out_shape=jax.ShapeDtypeStruct((FRAME, B), jnp.int32),
    )(alignT, textT)


def _pep_body(pe_ref, w_ref, bias_ref, out_ref):
    out_ref[...] = (
        jnp.dot(pe_ref[...], w_ref[...], preferred_element_type=jnp.float32)
        + bias_ref[...]
    )


def _compute_pep(pe, w_pos, bias_row):
    blk = 512
    return pl.pallas_call(
        _pep_body,
        grid=(FRAME // blk,),
        in_specs=[
            pl.BlockSpec((blk, D), lambda i: (i, 0)),
            pl.BlockSpec((D, D), lambda i: (0, 0)),
            pl.BlockSpec((1, D), lambda i: (0, 0)),
        ],
        out_specs=pl.BlockSpec((blk, D), lambda i: (i, 0)),
        out_shape=jax.ShapeDtypeStruct((FRAME, D), jnp.float32),
    )(pe, w_pos, bias_row)


def _main_body(enc_ref, w_ref, pep_ref, idx_ref, pitch_ref, beats_ref,
               wp_ref, de_ref, out_ref, s_ref):
    max_idx = idx_ref[0, 0, FRAME - 1]
    n_chunks = max_idx // MCH + 1

    def mm_chunk(c, _):
        enc = enc_ref[0, pl.ds(c * MCH, MCH), :]
        s_ref[pl.ds(c * MCH, MCH), :] = enc + jnp.dot(
            enc, w_ref[...], preferred_element_type=jnp.float32)
        return 0

    jax.lax.fori_loop(0, n_chunks, mm_chunk, 0)

    wp = wp_ref[...]
    de = de_ref[...]

    def group(g, _):
        i0 = idx_ref[0, 0, g * GRP]
        i7 = idx_ref[0, 0, g * GRP + GRP - 1]
        base = (pep_ref[pl.ds(g * GRP, GRP), :]
                + pitch_ref[0, pl.ds(g * GRP, GRP), :] * wp
                + beats_ref[0, pl.ds(g * GRP, GRP), :].astype(jnp.float32) * de)

        def uniform(_):
            return jnp.broadcast_to(s_ref[pl.ds(i0, 1), :], (GRP, D))

        def ragged(_):
            rows = [s_ref[pl.ds(idx_ref[0, 0, g * GRP + r], 1), :] for r in range(GRP)]
            return jnp.concatenate(rows, axis=0)

        s_tile = jax.lax.cond(i0 == i7, uniform, ragged, 0)
        out_ref[0, pl.ds(g * GRP, GRP), :] = base + s_tile
        return 0

    jax.lax.fori_loop(0, FRAME // GRP, group, 0)


def _main(enc, w_pos, pep, idx3, pitch, beats, w_pitch, de_row):
    return pl.pallas_call(
        _main_body,
        grid=(B,),
        in_specs=[
            pl.BlockSpec((1, TTEXT, D), lambda b: (b, 0, 0)),
            pl.BlockSpec((D, D), lambda b: (0, 0)),
            pl.BlockSpec((FRAME, D), lambda b: (0, 0)),
            pl.BlockSpec((1, 1, FRAME), lambda b: (b, 0, 0), memory_space=pltpu.SMEM),
            pl.BlockSpec((1, FRAME, 1), lambda b: (b, 0, 0)),
            pl.BlockSpec((1, FRAME, 1), lambda b: (b, 0, 0)),
            pl.BlockSpec((1, D), lambda b: (0, 0)),
            pl.BlockSpec((1, D), lambda b: (0, 0)),
        ],
        out_specs=pl.BlockSpec((1, FRAME, D), lambda b: (b, 0, 0)),
        out_shape=jax.ShapeDtypeStruct((B, FRAME, D), jnp.float32),
        scratch_shapes=[pltpu.VMEM((TTEXT, D), jnp.float32)],
    )(enc, w_pos, pep, idx3, pitch, beats, w_pitch, de_row)


def kernel(encoder_out, align_phone, text_phone, pitch, beats,
           W_pitch, b_pitch, W_pos, b_pos, emb_beats):
    align = align_phone.astype(jnp.int32)
    text = text_phone.astype(jnp.int32)
    alignT = align.at[:, 0].set(text[:, 0]).T
    textT = text.T
    idxT = _compute_idx(alignT, textT)
    idx3 = idxT.T.reshape(B, 1, FRAME)

    pe = jnp.asarray(_PE)
    bias_row = (b_pos + b_pitch + emb_beats[0]).reshape(1, D)
    de_row = (emb_beats[1] - emb_beats[0]).reshape(1, D)
    pep = _compute_pep(pe, W_pos, bias_row)

    return _main(encoder_out, W_pos, pep, idx3, pitch,
                 beats.astype(jnp.int32), W_pitch.reshape(1, D), de_row)


# speedup vs baseline: 24.3562x; 1.2760x over previous
import numpy as np
import jax
import jax.numpy as jnp
from jax.experimental import pallas as pl
from jax.experimental.pallas import tpu as pltpu

B, FRAME, TTEXT, D = 16, 2048, 512, 1024
SBLK = 64
GRP = 8
MCH = 64


def _pe_const(length, d):
    pos = np.arange(length, dtype=np.float32)[:, None]
    div = np.exp(np.arange(0, d, 2, dtype=np.float32) * (-np.log(10000.0) / d))
    pe = np.zeros((length, d), dtype=np.float32)
    pe[:, 0::2] = np.sin(pos * div)
    pe[:, 1::2] = np.cos(pos * div)
    return pe


_PE = _pe_const(FRAME, D)


def _scan_body(alignT_ref, textT_ref, idxT_ref):
    tmax = jnp.int32(TTEXT - 1)

    def gather_text(nind):
        def acc_chunk(m, acc):
            tv = textT_ref[pl.ds(m * 8, 8), :]
            rows = m * 8 + jax.lax.broadcasted_iota(jnp.int32, (8, B), 0)
            hit = rows == nind
            return acc + jnp.sum(jnp.where(hit, tv, 0), axis=0, keepdims=True)
        return jax.lax.fori_loop(0, TTEXT // 8, acc_chunk,
                                 jnp.zeros((1, B), jnp.int32))

    def slow_block(k, ind, bt):
        def step(j, carry):
            ind, bt = carry
            a = alignT_ref[pl.ds(k * SBLK + j, 1), :]
            adv = a != bt
            nind = jnp.where(adv, jnp.minimum(ind + 1, tmax), ind)
            nbt = jnp.where(adv, gather_text(nind), bt)
            idxT_ref[pl.ds(k * SBLK + j, 1), :] = nind
            return nind, nbt

        return jax.lax.fori_loop(0, SBLK, step, (ind, bt))

    def block(k, carry):
        ind, bt = carry
        blk = alignT_ref[pl.ds(k * SBLK, SBLK), :]
        uniform = jnp.all(blk == bt)

        def fast(args):
            ind, bt = args
            idxT_ref[pl.ds(k * SBLK, SBLK), :] = jnp.broadcast_to(ind, (SBLK, B))
            return ind, bt

        return jax.lax.cond(uniform, fast, lambda args: slow_block(k, *args),
                            (ind, bt))

    init = (jnp.zeros((1, B), jnp.int32), textT_ref[0:1, :])
    jax.lax.fori_loop(0, FRAME // SBLK, block, init)


def _compute_idx(alignT, textT):
    return pl.pallas_call(
        _scan_body,
        in_specs=[
            pl.BlockSpec((FRAME, B), lambda: (0, 0)),
            pl.BlockSpec((TTEXT, B), lambda: (0, 0)),
        ],
        out_specs=pl.BlockSpec((FRAME, B), lambda: (0, 0)),
        out_shape=jax.ShapeDtypeStruct((FRAME, B), jnp.int32),
    )(alignT, textT)


def _pep_body(pe_ref, w_ref, bias_ref, out_ref):
    out_ref[...] = (
        jnp.dot(pe_ref[...], w_ref[...], preferred_element_type=jnp.float32)
        + bias_ref[...]
    )


def _compute_pep(pe, w_pos, bias_row):
    blk = 512
    return pl.pallas_call(
        _pep_body,
        grid=(FRAME // blk,),
        in_specs=[
            pl.BlockSpec((blk, D), lambda i: (i, 0)),
            pl.BlockSpec((D, D), lambda i: (0, 0)),
            pl.BlockSpec((1, D), lambda i: (0, 0)),
        ],
        out_specs=pl.BlockSpec((blk, D), lambda i: (i, 0)),
        out_shape=jax.ShapeDtypeStruct((FRAME, D), jnp.float32),
    )(pe, w_pos, bias_row)


def _main_body(enc_ref, w_ref, pep_ref, idx_ref, pitch_ref, beats_ref,
               wp_ref, de_ref, out_ref, s_ref):
    max_idx = idx_ref[0, 0, FRAME - 1]
    n_chunks = max_idx // MCH + 1

    def mm_chunk(c, _):
        enc = enc_ref[0, pl.ds(c * MCH, MCH), :]
        s_ref[pl.ds(c * MCH, MCH), :] = enc + jnp.dot(
            enc, w_ref[...], preferred_element_type=jnp.float32)
        return 0

    jax.lax.fori_loop(0, n_chunks, mm_chunk, 0)

    wp = wp_ref[...]
    de = de_ref[...]

    def base_tile(g):
        return (pep_ref[pl.ds(g * GRP, GRP), :]
                + pitch_ref[0, pl.ds(g * GRP, GRP), :] * wp
                + beats_ref[0, pl.ds(g * GRP, GRP), :].astype(jnp.float32) * de)

    def group(g, _):
        i0 = idx_ref[0, 0, g * GRP]
        out_ref[0, pl.ds(g * GRP, GRP), :] = base_tile(g) + s_ref[pl.ds(i0, 1), :]
        return 0

    jax.lax.fori_loop(0, FRAME // GRP, group, 0, unroll=2)

    def fixup(g, _):
        i0 = idx_ref[0, 0, g * GRP]
        i7 = idx_ref[0, 0, g * GRP + GRP - 1]

        def redo(_):
            rows = [s_ref[pl.ds(idx_ref[0, 0, g * GRP + r], 1), :] for r in range(GRP)]
            out_ref[0, pl.ds(g * GRP, GRP), :] = (
                base_tile(g) + jnp.concatenate(rows, axis=0))
            return 0

        jax.lax.cond(i0 != i7, redo, lambda _: 0, 0)
        return 0

    jax.lax.fori_loop(0, FRAME // GRP, fixup, 0)


def _main(enc, w_pos, pep, idx3, pitch, beats, w_pitch, de_row):
    return pl.pallas_call(
        _main_body,
        grid=(B,),
        in_specs=[
            pl.BlockSpec((1, TTEXT, D), lambda b: (b, 0, 0)),
            pl.BlockSpec((D, D), lambda b: (0, 0)),
            pl.BlockSpec((FRAME, D), lambda b: (0, 0)),
            pl.BlockSpec((1, 1, FRAME), lambda b: (b, 0, 0), memory_space=pltpu.SMEM),
            pl.BlockSpec((1, FRAME, 1), lambda b: (b, 0, 0)),
            pl.BlockSpec((1, FRAME, 1), lambda b: (b, 0, 0)),
            pl.BlockSpec((1, D), lambda b: (0, 0)),
            pl.BlockSpec((1, D), lambda b: (0, 0)),
        ],
        out_specs=pl.BlockSpec((1, FRAME, D), lambda b: (b, 0, 0)),
        out_shape=jax.ShapeDtypeStruct((B, FRAME, D), jnp.float32),
        scratch_shapes=[pltpu.VMEM((TTEXT, D), jnp.float32)],
    )(enc, w_pos, pep, idx3, pitch, beats, w_pitch, de_row)


def kernel(encoder_out, align_phone, text_phone, pitch, beats,
           W_pitch, b_pitch, W_pos, b_pos, emb_beats):
    align = align_phone.astype(jnp.int32)
    text = text_phone.astype(jnp.int32)
    alignT = align.at[:, 0].set(text[:, 0]).T
    textT = text.T
    idxT = _compute_idx(alignT, textT)
    idx3 = idxT.T.reshape(B, 1, FRAME)

    pe = jnp.asarray(_PE)
    bias_row = (b_pos + b_pitch + emb_beats[0]).reshape(1, D)
    de_row = (emb_beats[1] - emb_beats[0]).reshape(1, D)
    pep = _compute_pep(pe, W_pos, bias_row)

    return _main(encoder_out, W_pos, pep, idx3, pitch,
                 beats.astype(jnp.int32), W_pitch.reshape(1, D), de_row)


# speedup vs baseline: 61.1288x; 2.5098x over previous
import numpy as np
import jax
import jax.numpy as jnp
from jax.experimental import pallas as pl
from jax.experimental.pallas import tpu as pltpu

B, FRAME, TTEXT, D = 16, 2048, 512, 1024
SBLK = 64
GRP = 8
MCH = 64


def _pe_const(length, d):
    pos = np.arange(length, dtype=np.float32)[:, None]
    div = np.exp(np.arange(0, d, 2, dtype=np.float32) * (-np.log(10000.0) / d))
    pe = np.zeros((length, d), dtype=np.float32)
    pe[:, 0::2] = np.sin(pos * div)
    pe[:, 1::2] = np.cos(pos * div)
    return pe


_PE = _pe_const(FRAME, D)


def _scan_body(alignT_ref, textT_ref, idxT_ref):
    tmax = jnp.int32(TTEXT - 1)

    def gather_text(nind):
        def acc_chunk(m, acc):
            tv = textT_ref[pl.ds(m * 8, 8), :]
            rows = m * 8 + jax.lax.broadcasted_iota(jnp.int32, (8, B), 0)
            hit = rows == nind
            return acc + jnp.sum(jnp.where(hit, tv, 0), axis=0, keepdims=True)
        return jax.lax.fori_loop(0, TTEXT // 8, acc_chunk,
                                 jnp.zeros((1, B), jnp.int32))

    def slow_block(k, ind, bt):
        def step(j, carry):
            ind, bt = carry
            a = alignT_ref[pl.ds(k * SBLK + j, 1), :]
            adv = a != bt
            nind = jnp.where(adv, jnp.minimum(ind + 1, tmax), ind)
            nbt = jnp.where(adv, gather_text(nind), bt)
            idxT_ref[pl.ds(k * SBLK + j, 1), :] = nind
            return nind, nbt

        return jax.lax.fori_loop(0, SBLK, step, (ind, bt))

    def block(k, carry):
        ind, bt = carry
        blk = alignT_ref[pl.ds(k * SBLK, SBLK), :]
        uniform = jnp.all(blk == bt)

        def fast(args):
            ind, bt = args
            idxT_ref[pl.ds(k * SBLK, SBLK), :] = jnp.broadcast_to(ind, (SBLK, B))
            return ind, bt

        return jax.lax.cond(uniform, fast, lambda args: slow_block(k, *args),
                            (ind, bt))

    init = (jnp.zeros((1, B), jnp.int32), textT_ref[0:1, :])
    jax.lax.fori_loop(0, FRAME // SBLK, block, init)


def _compute_idx(alignT, textT):
    return pl.pallas_call(
        _scan_body,
        in_specs=[
            pl.BlockSpec((FRAME, B), lambda: (0, 0)),
            pl.BlockSpec((TTEXT, B), lambda: (0, 0)),
        ],
        out_specs=pl.BlockSpec((FRAME, B), lambda: (0, 0)),
        out_shape=jax.ShapeDtypeStruct((FRAME, B), jnp.int32),
    )(alignT, textT)


def _pep_body(pe_ref, w_ref, bias_ref, out_ref):
    out_ref[...] = (
        jnp.dot(pe_ref[...], w_ref[...], preferred_element_type=jnp.float32)
        + bias_ref[...]
    )


def _compute_pep(pe, w_pos, bias_row):
    blk = 512
    return pl.pallas_call(
        _pep_body,
        grid=(FRAME // blk,),
        in_specs=[
            pl.BlockSpec((blk, D), lambda i: (i, 0)),
            pl.BlockSpec((D, D), lambda i: (0, 0)),
            pl.BlockSpec((1, D), lambda i: (0, 0)),
        ],
        out_specs=pl.BlockSpec((blk, D), lambda i: (i, 0)),
        out_shape=jax.ShapeDtypeStruct((FRAME, D), jnp.float32),
    )(pe, w_pos, bias_row)


def _main_body(enc_ref, w_ref, pep_ref, idx_ref, pb_ref, wde_ref,
               out_ref, s_ref, ptb_ref):
    max_idx = idx_ref[0, 0, FRAME - 1]
    n_chunks = max_idx // MCH + 1

    def mm_chunk(c, _):
        enc = enc_ref[0, pl.ds(c * MCH, MCH), :]
        s_ref[pl.ds(c * MCH, MCH), :] = enc + jnp.dot(
            enc, w_ref[...], preferred_element_type=jnp.float32)
        return 0

    jax.lax.fori_loop(0, n_chunks, mm_chunk, 0)

    ptb_ref[...] = pep_ref[...] + jnp.dot(
        pb_ref[0], wde_ref[...], preferred_element_type=jnp.float32)

    def group(g, _):
        i0 = idx_ref[0, 0, g * GRP]
        out_ref[0, pl.ds(g * GRP, GRP), :] = (
            ptb_ref[pl.ds(g * GRP, GRP), :] + s_ref[pl.ds(i0, 1), :])
        return 0

    jax.lax.fori_loop(0, FRAME // GRP, group, 0, unroll=2)

    CL = 512

    def fixup(g, _):
        i0 = idx_ref[0, 0, g * GRP]
        i7 = idx_ref[0, 0, g * GRP + GRP - 1]

        def redo(_):
            rows = [s_ref[pl.ds(idx_ref[0, 0, g * GRP + r], 1), :] for r in range(GRP)]
            out_ref[0, pl.ds(g * GRP, GRP), :] = (
                ptb_ref[pl.ds(g * GRP, GRP), :] + jnp.concatenate(rows, axis=0))
            return 0

        jax.lax.cond(i0 != i7, redo, lambda _: 0, 0)
        return 0

    def cluster(c, _):
        c0 = idx_ref[0, 0, c * CL]
        c1 = idx_ref[0, 0, c * CL + CL - 1]

        def dirty(_):
            jax.lax.fori_loop(c * (CL // GRP), (c + 1) * (CL // GRP), fixup, 0)
            return 0

        jax.lax.cond(c0 != c1, dirty, lambda _: 0, 0)
        return 0

    jax.lax.fori_loop(0, FRAME // CL, cluster, 0)


def _main(enc, w_pos, pep, idx3, pb8, wde8):
    return pl.pallas_call(
        _main_body,
        grid=(B,),
        in_specs=[
            pl.BlockSpec((1, TTEXT, D), lambda b: (b, 0, 0)),
            pl.BlockSpec((D, D), lambda b: (0, 0)),
            pl.BlockSpec((FRAME, D), lambda b: (0, 0)),
            pl.BlockSpec((1, 1, FRAME), lambda b: (b, 0, 0), memory_space=pltpu.SMEM),
            pl.BlockSpec((1, FRAME, 8), lambda b: (b, 0, 0)),
            pl.BlockSpec((8, D), lambda b: (0, 0)),
        ],
        out_specs=pl.BlockSpec((1, FRAME, D), lambda b: (b, 0, 0)),
        out_shape=jax.ShapeDtypeStruct((B, FRAME, D), jnp.float32),
        scratch_shapes=[pltpu.VMEM((TTEXT, D), jnp.float32),
                        pltpu.VMEM((FRAME, D), jnp.float32)],
    )(enc, w_pos, pep, idx3, pb8, wde8)


def kernel(encoder_out, align_phone, text_phone, pitch, beats,
           W_pitch, b_pitch, W_pos, b_pos, emb_beats):
    align = align_phone.astype(jnp.int32)
    text = text_phone.astype(jnp.int32)
    alignT = align.at[:, 0].set(text[:, 0]).T
    textT = text.T
    idxT = _compute_idx(alignT, textT)
    idx3 = idxT.T.reshape(B, 1, FRAME)

    pe = jnp.asarray(_PE)
    bias_row = (b_pos + b_pitch + emb_beats[0]).reshape(1, D)
    de_row = (emb_beats[1] - emb_beats[0]).reshape(1, D)
    pep = _compute_pep(pe, W_pos, bias_row)

    pb8 = jnp.concatenate(
        [pitch, beats.astype(jnp.float32),
         jnp.zeros((B, FRAME, 6), jnp.float32)], axis=-1)
    wde8 = jnp.concatenate(
        [W_pitch.reshape(1, D), de_row,
         jnp.zeros((6, D), jnp.float32)], axis=0)

    return _main(encoder_out, W_pos, pep, idx3, pb8, wde8)


# speedup vs baseline: 64.3667x; 1.0530x over previous
import numpy as np
import jax
import jax.numpy as jnp
from jax.experimental import pallas as pl
from jax.experimental.pallas import tpu as pltpu

B, FRAME, TTEXT, D = 16, 2048, 512, 1024
SBLK = 64
GRP = 8
MCH = 64


def _pe_const(length, d):
    pos = np.arange(length, dtype=np.float32)[:, None]
    div = np.exp(np.arange(0, d, 2, dtype=np.float32) * (-np.log(10000.0) / d))
    pe = np.zeros((length, d), dtype=np.float32)
    pe[:, 0::2] = np.sin(pos * div)
    pe[:, 1::2] = np.cos(pos * div)
    return pe


_PE = _pe_const(FRAME, D)


def _scan_body(alignT_ref, textT_ref, idxT_ref):
    tmax = jnp.int32(TTEXT - 1)

    def gather_text(nind):
        def acc_chunk(m, acc):
            tv = textT_ref[pl.ds(m * 8, 8), :]
            rows = m * 8 + jax.lax.broadcasted_iota(jnp.int32, (8, B), 0)
            hit = rows == nind
            return acc + jnp.sum(jnp.where(hit, tv, 0), axis=0, keepdims=True)
        return jax.lax.fori_loop(0, TTEXT // 8, acc_chunk,
                                 jnp.zeros((1, B), jnp.int32))

    def slow_block(k, ind, bt):
        def step(j, carry):
            ind, bt = carry
            a = alignT_ref[pl.ds(k * SBLK + j, 1), :]
            adv = a != bt
            nind = jnp.where(adv, jnp.minimum(ind + 1, tmax), ind)
            nbt = jnp.where(adv, gather_text(nind), bt)
            idxT_ref[pl.ds(k * SBLK + j, 1), :] = nind
            return nind, nbt

        return jax.lax.fori_loop(0, SBLK, step, (ind, bt))

    def block(k, carry):
        ind, bt = carry
        blk = alignT_ref[pl.ds(k * SBLK, SBLK), :]
        uniform = jnp.all(blk == bt)

        def fast(args):
            ind, bt = args
            idxT_ref[pl.ds(k * SBLK, SBLK), :] = jnp.broadcast_to(ind, (SBLK, B))
            return ind, bt

        return jax.lax.cond(uniform, fast, lambda args: slow_block(k, *args),
                            (ind, bt))

    init = (jnp.zeros((1, B), jnp.int32), textT_ref[0:1, :])
    jax.lax.fori_loop(0, FRAME // SBLK, block, init)


def _compute_idx(alignT, textT):
    return pl.pallas_call(
        _scan_body,
        in_specs=[
            pl.BlockSpec((FRAME, B), lambda: (0, 0)),
            pl.BlockSpec((TTEXT, B), lambda: (0, 0)),
        ],
        out_specs=pl.BlockSpec((FRAME, B), lambda: (0, 0)),
        out_shape=jax.ShapeDtypeStruct((FRAME, B), jnp.int32),
    )(alignT, textT)


def _main_body(enc_ref, w_ref, pe_ref, bias_ref, idx_ref, pb_ref, wde_ref,
               out_ref, s_ref, ptb_ref, pep_ref):
    @pl.when(pl.program_id(0) == 0)
    def _():
        for ch in range(4):
            pep_ref[ch * 512:(ch + 1) * 512, :] = (
                jnp.dot(pe_ref[ch * 512:(ch + 1) * 512, :], w_ref[...],
                        preferred_element_type=jnp.float32)
                + bias_ref[...])

    max_idx = idx_ref[0, 0, FRAME - 1]
    n_chunks = max_idx // MCH + 1

    def mm_chunk(c, _):
        enc = enc_ref[0, pl.ds(c * MCH, MCH), :]
        s_ref[pl.ds(c * MCH, MCH), :] = enc + jnp.dot(
            enc, w_ref[...], preferred_element_type=jnp.float32)
        return 0

    jax.lax.fori_loop(0, n_chunks, mm_chunk, 0)

    ptb_ref[...] = pep_ref[...] + jnp.dot(
        pb_ref[0], wde_ref[...], preferred_element_type=jnp.float32)

    def group(g, _):
        i0 = idx_ref[0, 0, g * GRP]
        out_ref[0, pl.ds(g * GRP, GRP), :] = (
            ptb_ref[pl.ds(g * GRP, GRP), :] + s_ref[pl.ds(i0, 1), :])
        return 0

    jax.lax.fori_loop(0, FRAME // GRP, group, 0, unroll=2)

    CL = 512

    def fixup(g, _):
        i0 = idx_ref[0, 0, g * GRP]
        i7 = idx_ref[0, 0, g * GRP + GRP - 1]

        def redo(_):
            rows = [s_ref[pl.ds(idx_ref[0, 0, g * GRP + r], 1), :] for r in range(GRP)]
            out_ref[0, pl.ds(g * GRP, GRP), :] = (
                ptb_ref[pl.ds(g * GRP, GRP), :] + jnp.concatenate(rows, axis=0))
            return 0

        jax.lax.cond(i0 != i7, redo, lambda _: 0, 0)
        return 0

    def cluster(c, _):
        c0 = idx_ref[0, 0, c * CL]
        c1 = idx_ref[0, 0, c * CL + CL - 1]

        def dirty(_):
            jax.lax.fori_loop(c * (CL // GRP), (c + 1) * (CL // GRP), fixup, 0)
            return 0

        jax.lax.cond(c0 != c1, dirty, lambda _: 0, 0)
        return 0

    jax.lax.fori_loop(0, FRAME // CL, cluster, 0)


def _main(enc, w_pos, pe, bias_row, idx3, pb8, wde8):
    return pl.pallas_call(
        _main_body,
        grid=(B,),
        in_specs=[
            pl.BlockSpec((1, TTEXT, D), lambda b: (b, 0, 0)),
            pl.BlockSpec((D, D), lambda b: (0, 0)),
            pl.BlockSpec((FRAME, D), lambda b: (0, 0)),
            pl.BlockSpec((1, D), lambda b: (0, 0)),
            pl.BlockSpec((1, 1, FRAME), lambda b: (b, 0, 0), memory_space=pltpu.SMEM),
            pl.BlockSpec((1, FRAME, 8), lambda b: (b, 0, 0)),
            pl.BlockSpec((8, D), lambda b: (0, 0)),
        ],
        out_specs=pl.BlockSpec((1, FRAME, D), lambda b: (b, 0, 0)),
        out_shape=jax.ShapeDtypeStruct((B, FRAME, D), jnp.float32),
        scratch_shapes=[pltpu.VMEM((TTEXT, D), jnp.float32),
                        pltpu.VMEM((FRAME, D), jnp.float32),
                        pltpu.VMEM((FRAME, D), jnp.float32)],
    )(enc, w_pos, pe, bias_row, idx3, pb8, wde8)


def kernel(encoder_out, align_phone, text_phone, pitch, beats,
           W_pitch, b_pitch, W_pos, b_pos, emb_beats):
    align = align_phone.astype(jnp.int32)
    text = text_phone.astype(jnp.int32)
    alignT = align.at[:, 0].set(text[:, 0]).T
    textT = text.T
    idxT = _compute_idx(alignT, textT)
    idx3 = idxT.T.reshape(B, 1, FRAME)

    pe = jnp.asarray(_PE)
    bias_row = (b_pos + b_pitch + emb_beats[0]).reshape(1, D)
    de_row = (emb_beats[1] - emb_beats[0]).reshape(1, D)

    pb8 = jnp.concatenate(
        [pitch, beats.astype(jnp.float32),
         jnp.zeros((B, FRAME, 6), jnp.float32)], axis=-1)
    wde8 = jnp.concatenate(
        [W_pitch.reshape(1, D), de_row,
         jnp.zeros((6, D), jnp.float32)], axis=0)

    return _main(encoder_out, W_pos, pe, bias_row, idx3, pb8, wde8)
